# baseline (speedup 1.0000x reference)
import functools
import math

import jax
import jax.numpy as jnp
from jax import lax
from jax.experimental import pallas as pl
from jax.experimental.pallas import tpu as pltpu
from jax.experimental.pallas import tpu_sc as plsc

N = 50000
B = 500
H = 128
E = 800000
NB = 1000
GRID_N = N // NB

CW = 32
EB = 128
E_PAD = 802816
EROWS = E_PAD // EB
TROWS = EROWS // 16
GB = 56
NBLK = TROWS // GB
NBUF = 4
NQ = GB // NBUF
NACC = 50048
ZR = NACC // 16


def _sc_segsum_body(src_hbm, dst_hbm, zeros_hbm, m0, m1, m2, m3,
                    a0, a1, a2, a3,
                    src_blk, dst_blk, rows, acc, *sems):
    gsem = sems[:NBUF]
    ssem = sems[NBUF:]
    c = lax.axis_index("c")
    s = lax.axis_index("s")

    def process(m_ref, a_ref):
        pltpu.sync_copy(zeros_hbm.at[pl.ds(s * ZR, ZR)], acc.at[pl.ds(s * ZR, ZR)])
        plsc.subcore_barrier()

        def blk_body(blk, carry):
            @pl.when(blk > 0)
            def _drain():
                for b in range(NBUF):
                    pltpu.make_async_copy(
                        rows.at[b], acc.at[dst_blk.at[b]], ssem[b]).wait()

            base = s * TROWS + blk * GB
            pltpu.sync_copy(src_hbm.at[pl.ds(base, GB)], src_blk)
            pltpu.sync_copy(dst_hbm.at[pl.ds(base, GB)], dst_blk)
            for gg in range(NQ):
                for b in range(NBUF):
                    r = gg * NBUF + b
                    if gg > 0:
                        pltpu.make_async_copy(
                            rows.at[b], acc.at[dst_blk.at[r]], ssem[b]).wait()
                    pltpu.async_copy(m_ref.at[src_blk.at[r]], rows.at[b], gsem[b])
                for b in range(NBUF):
                    r = gg * NBUF + b
                    pltpu.make_async_copy(
                        m_ref.at[src_blk.at[r]], rows.at[b], gsem[b]).wait()
                    pltpu.async_copy(rows.at[b], acc.at[dst_blk.at[r]], ssem[b],
                                     add=True)
            return carry

        lax.fori_loop(0, NBLK, blk_body, 0)
        for b in range(NBUF):
            pltpu.make_async_copy(
                rows.at[b], acc.at[dst_blk.at[b]], ssem[b]).wait()
        plsc.subcore_barrier()
        pltpu.sync_copy(acc.at[pl.ds(s * ZR, ZR)], a_ref.at[pl.ds(s * ZR, ZR)])
        plsc.subcore_barrier()

    for cc in range(2):
        @pl.when(c == 0)
        def _c0():
            process((m0, m1)[cc], (a0, a1)[cc])

        @pl.when(c == 1)
        def _c1():
            process((m2, m3)[cc], (a2, a3)[cc])


def _sc_segsum(src2d, dst2d, zeros, m_chunks):
    mesh = plsc.VectorSubcoreMesh(core_axis_name="c", subcore_axis_name="s")
    out_type = [jax.ShapeDtypeStruct((NACC, CW), jnp.float32) for _ in range(4)]
    scratch = [
        pltpu.VMEM((GB, EB), jnp.int32),
        pltpu.VMEM((GB, EB), jnp.int32),
        pltpu.VMEM((NBUF, EB, CW), jnp.float32),
        pltpu.VMEM_SHARED((NACC, CW), jnp.float32),
    ] + [pltpu.SemaphoreType.DMA] * (2 * NBUF)
    f = pl.kernel(_sc_segsum_body, out_type=out_type, mesh=mesh,
                  scratch_types=scratch,
                  compiler_params=pltpu.CompilerParams(use_tc_tiling_on_sc=False))
    return f(src2d, dst2d, zeros, *m_chunks)


def _mm_bias_kernel(x_ref, w_ref, b_ref, o_ref, *, act):
    y = jnp.dot(x_ref[...], w_ref[...], preferred_element_type=jnp.float32) + b_ref[...]
    if act == "relu":
        y = jnp.maximum(y, 0.0)
    o_ref[...] = y


def _inproj_kernel(x_ref, tf_ref, df_ref, w_ref, wtd_ref, b_ref, o_ref):
    y = jnp.dot(x_ref[...], w_ref[...], preferred_element_type=jnp.float32)
    y = y + tf_ref[...] * wtd_ref[0:1, :] + df_ref[...] * wtd_ref[1:2, :]
    o_ref[...] = jnp.maximum(y + b_ref[...], 0.0)


def _inproj(x_emb, tf, df, w_t, wtd, b):
    return pl.pallas_call(
        _inproj_kernel,
        grid=(GRID_N,),
        in_specs=[
            pl.BlockSpec((NB, H), lambda i: (i, 0)),
            pl.BlockSpec((NB, 1), lambda i: (i, 0)),
            pl.BlockSpec((NB, 1), lambda i: (i, 0)),
            pl.BlockSpec((H, H), lambda i: (0, 0)),
            pl.BlockSpec((2, H), lambda i: (0, 0)),
            pl.BlockSpec((1, H), lambda i: (0, 0)),
        ],
        out_specs=pl.BlockSpec((NB, H), lambda i: (i, 0)),
        out_shape=jax.ShapeDtypeStruct((N, H), jnp.float32),
    )(x_emb, tf, df, w_t, wtd, b.reshape(1, H))


def _mm_bias(x, w_t, b, act="none"):
    K = x.shape[1]
    M = w_t.shape[1]
    return pl.pallas_call(
        functools.partial(_mm_bias_kernel, act=act),
        grid=(GRID_N,),
        in_specs=[
            pl.BlockSpec((NB, K), lambda i: (i, 0)),
            pl.BlockSpec((K, M), lambda i: (0, 0)),
            pl.BlockSpec((1, M), lambda i: (0, 0)),
        ],
        out_specs=pl.BlockSpec((NB, M), lambda i: (i, 0)),
        out_shape=jax.ShapeDtypeStruct((N, M), jnp.float32),
    )(x, w_t, b.reshape(1, M))


def _mm4_kernel(x_ref, w_ref, o0, o1, o2, o3):
    y = jnp.dot(x_ref[...], w_ref[...], preferred_element_type=jnp.float32)
    for i, o in enumerate((o0, o1, o2, o3)):
        o[...] = y[:, i * CW:(i + 1) * CW]


def _mm4(x, w):
    return pl.pallas_call(
        _mm4_kernel,
        grid=(GRID_N,),
        in_specs=[
            pl.BlockSpec((NB, H), lambda i: (i, 0)),
            pl.BlockSpec((H, H), lambda i: (0, 0)),
        ],
        out_specs=[pl.BlockSpec((NB, CW), lambda i: (i, 0)) for _ in range(4)],
        out_shape=[jax.ShapeDtypeStruct((N, CW), jnp.float32) for _ in range(4)],
    )(x, w)


def _gru_kernel(a0, a1, a2, a3, x_ref, wih_ref, whh_ref, bih_ref, bhh_ref, o_ref):
    agg = jnp.concatenate([a0[...], a1[...], a2[...], a3[...]], axis=1)
    gi = jnp.dot(agg, wih_ref[...], preferred_element_type=jnp.float32) + bih_ref[...]
    gh = jnp.dot(x_ref[...], whh_ref[...], preferred_element_type=jnp.float32) + bhh_ref[...]
    i_r, i_z, i_n = gi[:, :H], gi[:, H:2 * H], gi[:, 2 * H:]
    h_r, h_z, h_n = gh[:, :H], gh[:, H:2 * H], gh[:, 2 * H:]
    r = jax.nn.sigmoid(i_r + h_r)
    z = jax.nn.sigmoid(i_z + h_z)
    n = jnp.tanh(i_n + r * h_n)
    o_ref[...] = (1.0 - z) * n + z * x_ref[...]


def _gru(agg_chunks, x, wih_t, whh_t, bih, bhh):
    return pl.pallas_call(
        _gru_kernel,
        grid=(GRID_N,),
        in_specs=[
            pl.BlockSpec((NB, CW), lambda i: (i, 0)),
            pl.BlockSpec((NB, CW), lambda i: (i, 0)),
            pl.BlockSpec((NB, CW), lambda i: (i, 0)),
            pl.BlockSpec((NB, CW), lambda i: (i, 0)),
            pl.BlockSpec((NB, H), lambda i: (i, 0)),
            pl.BlockSpec((H, 3 * H), lambda i: (0, 0)),
            pl.BlockSpec((H, 3 * H), lambda i: (0, 0)),
            pl.BlockSpec((1, 3 * H), lambda i: (0, 0)),
            pl.BlockSpec((1, 3 * H), lambda i: (0, 0)),
        ],
        out_specs=pl.BlockSpec((NB, H), lambda i: (i, 0)),
        out_shape=jax.ShapeDtypeStruct((N, H), jnp.float32),
    )(*agg_chunks, x, wih_t, whh_t, bih.reshape(1, 3 * H), bhh.reshape(1, 3 * H))


def _attn_w_kernel(qe_ref, k_ref, v_ref, m_ref, we_ref):
    s = jnp.dot(qe_ref[...] * k_ref[...], m_ref[...],
                preferred_element_type=jnp.float32) * (1.0 / math.sqrt(32.0))
    e = jnp.exp(s)
    we_ref[...] = jnp.concatenate([e * v_ref[...], e], axis=1)


def _attn_weights(qexp, k, v, headmask):
    return pl.pallas_call(
        _attn_w_kernel,
        grid=(GRID_N,),
        in_specs=[
            pl.BlockSpec((NB, H), lambda i: (i, 0)),
            pl.BlockSpec((NB, H), lambda i: (i, 0)),
            pl.BlockSpec((NB, H), lambda i: (i, 0)),
            pl.BlockSpec((H, H), lambda i: (0, 0)),
        ],
        out_specs=pl.BlockSpec((NB, 2 * H), lambda i: (i, 0)),
        out_shape=jax.ShapeDtypeStruct((NP, 2 * H), jnp.float32),
    )(qexp, k, v, headmask)


NP = 65536
WNODES = NP // 32
SB = 128
NBAT = WNODES // SB
BROWS = NP // SB
BACC = 512


def _sc_attnsum_body(we_hbm, bat_hbm, zb_hbm, out0, out1,
                     bat_blk, ubuf, acc, sem0, sem1, *ssems):
    c = lax.axis_index("c")
    s = lax.axis_index("s")
    w = c * 16 + s
    zr = BACC // 16
    pltpu.sync_copy(zb_hbm.at[pl.ds(s * zr, zr)], acc.at[pl.ds(s * zr, zr)])
    pltpu.sync_copy(bat_hbm.at[pl.ds(w * NBAT, NBAT)], bat_blk)
    plsc.subcore_barrier()
    usems = (sem0, sem1)

    def _load(j):
        pltpu.async_copy(
            we_hbm.at[pl.ds(w * WNODES + j * SB, SB)], ubuf.at[j % 2],
            usems[j % 2])

    _load(0)
    for j in range(NBAT):
        b = j % 2
        if j + 1 < NBAT:
            if j >= 1:
                pltpu.make_async_copy(
                    ubuf.at[1 - b], acc.at[bat_blk.at[j]], ssems[1 - b]).wait()
            _load(j + 1)
        pltpu.make_async_copy(
            we_hbm.at[pl.ds(w * WNODES + j * SB, SB)], ubuf.at[b],
            usems[b]).wait()
        pltpu.async_copy(ubuf.at[b], acc.at[bat_blk.at[j]], ssems[b],
                         add=True)
    for b in range(2):
        pltpu.make_async_copy(ubuf.at[b], acc.at[bat_blk.at[b]], ssems[b]).wait()
    plsc.subcore_barrier()

    @pl.when(c == 0)
    def _w0():
        pltpu.sync_copy(acc.at[pl.ds(s * zr, zr)], out0.at[pl.ds(s * zr, zr)])

    @pl.when(c == 1)
    def _w1():
        pltpu.sync_copy(acc.at[pl.ds(s * zr, zr)], out1.at[pl.ds(s * zr, zr)])


GROWS = NP // 128


def _sc_gather_body(tab_hbm, idx_hbm, out_hbm, idx_blk, gbuf, g0, g1, s0, s1):
    c = lax.axis_index("c")
    s = lax.axis_index("s")
    w = c * 16 + s
    gsems = (g0, g1)
    osems = (s0, s1)
    nb = GROWS // 32
    pltpu.sync_copy(idx_hbm.at[pl.ds(w * nb, nb)], idx_blk)

    def _g(j):
        pltpu.async_copy(tab_hbm.at[idx_blk.at[j]], gbuf.at[j % 2], gsems[j % 2])

    def _dst(j):
        return out_hbm.at[pl.ds((w * nb + j) * 128, 128)]

    _g(0)
    for j in range(nb):
        b = j % 2
        if j + 1 < nb:
            if j >= 1:
                pltpu.make_async_copy(gbuf.at[1 - b], _dst(j), osems[1 - b]).wait()
            _g(j + 1)
        pltpu.make_async_copy(tab_hbm.at[idx_blk.at[j]], gbuf.at[b],
                              gsems[b]).wait()
        pltpu.async_copy(gbuf.at[b], _dst(j), osems[b])
    for b in range(2):
        pltpu.make_async_copy(gbuf.at[b], _dst(b), osems[b]).wait()


def _sc_gather(table, idx2d):
    mesh = plsc.VectorSubcoreMesh(core_axis_name="c", subcore_axis_name="s")
    out_type = jax.ShapeDtypeStruct((NP, H), jnp.float32)
    scratch = [
        pltpu.VMEM((GROWS // 32, 128), jnp.int32),
        pltpu.VMEM((2, 128, H), jnp.float32),
        pltpu.SemaphoreType.DMA,
        pltpu.SemaphoreType.DMA,
        pltpu.SemaphoreType.DMA,
        pltpu.SemaphoreType.DMA,
    ]
    f = pl.kernel(_sc_gather_body, out_type=out_type, mesh=mesh,
                  scratch_types=scratch,
                  compiler_params=pltpu.CompilerParams(use_tc_tiling_on_sc=True))
    return f(table, idx2d)


def _sc_attnsum(we, bat2d, zb):
    mesh = plsc.VectorSubcoreMesh(core_axis_name="c", subcore_axis_name="s")
    out_type = [jax.ShapeDtypeStruct((BACC, 2 * H), jnp.float32) for _ in range(2)]
    scratch = [
        pltpu.VMEM((NBAT, SB), jnp.int32),
        pltpu.VMEM((2, SB, 2 * H), jnp.float32),
        pltpu.VMEM_SHARED((BACC, 2 * H), jnp.float32),
        pltpu.SemaphoreType.DMA,
        pltpu.SemaphoreType.DMA,
        pltpu.SemaphoreType.DMA,
        pltpu.SemaphoreType.DMA,
    ]
    f = pl.kernel(_sc_attnsum_body, out_type=out_type, mesh=mesh,
                  scratch_types=scratch,
                  compiler_params=pltpu.CompilerParams(use_tc_tiling_on_sc=False))
    return f(we, bat2d, zb)


def _head_kernel(a0_ref, a1_ref, wo_ref, bo_ref, o_ref):
    t = a0_ref[...] + a1_ref[...]
    es = t[:B, H:]
    ctx = t[:B, :H] / jnp.where(es > 0.0, es, 1.0)
    o_ref[...] = jnp.dot(ctx, wo_ref[...], preferred_element_type=jnp.float32) + bo_ref[...]


def _head(acc0, acc1, wo_t, bo):
    return pl.pallas_call(
        _head_kernel,
        in_specs=[
            pl.BlockSpec((BACC, 2 * H), lambda: (0, 0)),
            pl.BlockSpec((BACC, 2 * H), lambda: (0, 0)),
            pl.BlockSpec((H, H), lambda: (0, 0)),
            pl.BlockSpec((1, H), lambda: (0, 0)),
        ],
        out_specs=pl.BlockSpec((B, H), lambda: (0, 0)),
        out_shape=jax.ShapeDtypeStruct((B, H), jnp.float32),
    )(acc0, acc1, wo_t, bo.reshape(1, H))


def _logits_kernel(f_ref, w_ref, b_ref, o_ref):
    o_ref[...] = jnp.dot(f_ref[...], w_ref[...], preferred_element_type=jnp.float32) + b_ref[...]


def _logits(final, lin_w_t, lin_b):
    V = lin_w_t.shape[1]
    CB = 1280
    grid = (V + CB - 1) // CB
    return pl.pallas_call(
        _logits_kernel,
        grid=(grid,),
        in_specs=[
            pl.BlockSpec((B, H), lambda j: (0, 0)),
            pl.BlockSpec((H, CB), lambda j: (0, j)),
            pl.BlockSpec((1, CB), lambda j: (0, j)),
        ],
        out_specs=pl.BlockSpec((B, CB), lambda j: (0, j)),
        out_shape=jax.ShapeDtypeStruct((B, V), jnp.float32),
    )(final, lin_w_t, lin_b.reshape(1, V))


def kernel(node_idx, time_feat, dwell_feat, edge_index, batch, embed, W_in, b_in,
           ggnn_w, gru_w_ih, gru_w_hh, gru_b_ih, gru_b_hh,
           attn_in_w, attn_in_b, attn_out_w, attn_out_b, lin_w, lin_b):
    table = jnp.pad(embed, ((0, 0), (0, 2)))
    idx2d = jnp.concatenate(
        [node_idx.astype(jnp.int32), jnp.zeros((NP - N,), jnp.int32)]).reshape(GROWS, 128)
    x_emb = _sc_gather(table, idx2d)
    wtd = W_in.T[126:128, :]
    x = _inproj(x_emb, time_feat, dwell_feat, W_in.T, wtd, b_in)

    npad = E_PAD - E
    src2d = jnp.concatenate(
        [edge_index[0], jnp.zeros((npad,), edge_index.dtype)]).reshape(EROWS, EB)
    dst2d = jnp.concatenate(
        [edge_index[1], jnp.full((npad,), N, edge_index.dtype)]).reshape(EROWS, EB)
    zeros = jnp.zeros((NACC, CW), jnp.float32)
    for i in range(2):
        m_chunks = _mm4(x, ggnn_w[i])
        agg_chunks = _sc_segsum(src2d, dst2d, zeros, m_chunks)
        x = _gru(agg_chunks, x, gru_w_ih.T, gru_w_hh.T, gru_b_ih, gru_b_hh)

    qkv = _mm_bias(x, attn_in_w.T, attn_in_b)
    q = qkv[:, :H]
    k = qkv[:, H:2 * H]
    v = qkv[:, 2 * H:]
    lengths = jnp.bincount(batch, length=B)
    last_flat = jnp.cumsum(lengths) - 1
    qexp = jnp.take(jnp.take(q, last_flat, axis=0), batch, axis=0)
    headmask = jnp.repeat(jnp.eye(4, dtype=jnp.float32), 32, axis=0)
    headmask = jnp.repeat(headmask, 32, axis=1)
    we = _attn_weights(qexp, k, v, headmask)
    bat2d = jnp.concatenate(
        [batch.astype(jnp.int32), jnp.full((NP - N,), B, jnp.int32)]).reshape(BROWS, SB)
    zb = jnp.zeros((BACC, 2 * H), jnp.float32)
    acc0, acc1 = _sc_attnsum(we, bat2d, zb)
    final = _head(acc0, acc1, attn_out_w.T, attn_out_b)
    return _logits(final, lin_w.T, lin_b)


# speedup vs baseline: 1.2992x; 1.2992x over previous
import functools
import math

import jax
import jax.numpy as jnp
from jax import lax
from jax.experimental import pallas as pl
from jax.experimental.pallas import tpu as pltpu
from jax.experimental.pallas import tpu_sc as plsc

N = 50000
B = 500
H = 128
E = 800000
NB = 1000
GRID_N = N // NB

CW = 32
EB = 128
E_PAD = 802816
EROWS = E_PAD // EB
TROWS = EROWS // 16
GB = 56
NBLK = TROWS // GB
NBUF = 4
NQ = GB // NBUF
NACC = 50048
ZR = NACC // 16


def _sc_segsum_body(src_hbm, dst_hbm, zeros_hbm, m0, m1, m2, m3,
                    a0, a1, a2, a3,
                    src_blk, dst_blk, rows, acc, *sems):
    gsem = sems[:NBUF]
    ssem = sems[NBUF:]
    c = lax.axis_index("c")
    s = lax.axis_index("s")

    def process(m_ref, a_ref):
        pltpu.sync_copy(zeros_hbm.at[pl.ds(s * ZR, ZR)], acc.at[pl.ds(s * ZR, ZR)])
        plsc.subcore_barrier()

        def blk_body(blk, carry):
            @pl.when(blk > 0)
            def _drain():
                for b in range(NBUF):
                    pltpu.make_async_copy(
                        rows.at[b], acc.at[dst_blk.at[b]], ssem[b]).wait()

            base = s * TROWS + blk * GB
            pltpu.sync_copy(src_hbm.at[pl.ds(base, GB)], src_blk)
            pltpu.sync_copy(dst_hbm.at[pl.ds(base, GB)], dst_blk)
            for gg in range(NQ):
                for b in range(NBUF):
                    r = gg * NBUF + b
                    if gg > 0:
                        pltpu.make_async_copy(
                            rows.at[b], acc.at[dst_blk.at[r]], ssem[b]).wait()
                    pltpu.async_copy(m_ref.at[src_blk.at[r]], rows.at[b], gsem[b])
                for b in range(NBUF):
                    r = gg * NBUF + b
                    pltpu.make_async_copy(
                        m_ref.at[src_blk.at[r]], rows.at[b], gsem[b]).wait()
                    pltpu.async_copy(rows.at[b], acc.at[dst_blk.at[r]], ssem[b],
                                     add=True)
            return carry

        lax.fori_loop(0, NBLK, blk_body, 0)
        for b in range(NBUF):
            pltpu.make_async_copy(
                rows.at[b], acc.at[dst_blk.at[b]], ssem[b]).wait()
        plsc.subcore_barrier()
        pltpu.sync_copy(acc.at[pl.ds(s * ZR, ZR)], a_ref.at[pl.ds(s * ZR, ZR)])
        plsc.subcore_barrier()

    for cc in range(2):
        @pl.when(c == 0)
        def _c0():
            process((m0, m1)[cc], (a0, a1)[cc])

        @pl.when(c == 1)
        def _c1():
            process((m2, m3)[cc], (a2, a3)[cc])


def _sc_segsum(src2d, dst2d, zeros, m_chunks):
    mesh = plsc.VectorSubcoreMesh(core_axis_name="c", subcore_axis_name="s")
    out_type = [jax.ShapeDtypeStruct((NACC, CW), jnp.float32) for _ in range(4)]
    scratch = [
        pltpu.VMEM((GB, EB), jnp.int32),
        pltpu.VMEM((GB, EB), jnp.int32),
        pltpu.VMEM((NBUF, EB, CW), jnp.float32),
        pltpu.VMEM_SHARED((NACC, CW), jnp.float32),
    ] + [pltpu.SemaphoreType.DMA] * (2 * NBUF)
    f = pl.kernel(_sc_segsum_body, out_type=out_type, mesh=mesh,
                  scratch_types=scratch,
                  compiler_params=pltpu.CompilerParams(use_tc_tiling_on_sc=False))
    return f(src2d, dst2d, zeros, *m_chunks)


def _mm_bias_kernel(x_ref, w_ref, b_ref, o_ref, *, act):
    y = jnp.dot(x_ref[...], w_ref[...], preferred_element_type=jnp.float32) + b_ref[...]
    if act == "relu":
        y = jnp.maximum(y, 0.0)
    o_ref[...] = y


def _inproj_kernel(x_ref, tf_ref, df_ref, w_ref, wtd_ref, b_ref, o_ref):
    y = jnp.dot(x_ref[...], w_ref[...], preferred_element_type=jnp.float32)
    y = y + tf_ref[...] * wtd_ref[0:1, :] + df_ref[...] * wtd_ref[1:2, :]
    o_ref[...] = jnp.maximum(y + b_ref[...], 0.0)


def _inproj(x_emb, tf, df, w_t, wtd, b):
    K = x_emb.shape[1]
    return pl.pallas_call(
        _inproj_kernel,
        grid=(GRID_N,),
        in_specs=[
            pl.BlockSpec((NB, K), lambda i: (i, 0)),
            pl.BlockSpec((NB, 1), lambda i: (i, 0)),
            pl.BlockSpec((NB, 1), lambda i: (i, 0)),
            pl.BlockSpec((K, H), lambda i: (0, 0)),
            pl.BlockSpec((2, H), lambda i: (0, 0)),
            pl.BlockSpec((1, H), lambda i: (0, 0)),
        ],
        out_specs=pl.BlockSpec((NB, H), lambda i: (i, 0)),
        out_shape=jax.ShapeDtypeStruct((N, H), jnp.float32),
    )(x_emb, tf, df, w_t, wtd, b.reshape(1, H))


def _mm_bias(x, w_t, b, act="none"):
    K = x.shape[1]
    M = w_t.shape[1]
    return pl.pallas_call(
        functools.partial(_mm_bias_kernel, act=act),
        grid=(GRID_N,),
        in_specs=[
            pl.BlockSpec((NB, K), lambda i: (i, 0)),
            pl.BlockSpec((K, M), lambda i: (0, 0)),
            pl.BlockSpec((1, M), lambda i: (0, 0)),
        ],
        out_specs=pl.BlockSpec((NB, M), lambda i: (i, 0)),
        out_shape=jax.ShapeDtypeStruct((N, M), jnp.float32),
    )(x, w_t, b.reshape(1, M))


def _mm4_kernel(x_ref, w_ref, o0, o1, o2, o3):
    y = jnp.dot(x_ref[...], w_ref[...], preferred_element_type=jnp.float32)
    for i, o in enumerate((o0, o1, o2, o3)):
        o[...] = y[:, i * CW:(i + 1) * CW]


def _mm4(x, w):
    return pl.pallas_call(
        _mm4_kernel,
        grid=(GRID_N,),
        in_specs=[
            pl.BlockSpec((NB, H), lambda i: (i, 0)),
            pl.BlockSpec((H, H), lambda i: (0, 0)),
        ],
        out_specs=[pl.BlockSpec((NB, CW), lambda i: (i, 0)) for _ in range(4)],
        out_shape=[jax.ShapeDtypeStruct((N, CW), jnp.float32) for _ in range(4)],
    )(x, w)


def _gru_kernel(a0, a1, a2, a3, x_ref, wih_ref, whh_ref, bih_ref, bhh_ref, o_ref):
    agg = jnp.concatenate([a0[...], a1[...], a2[...], a3[...]], axis=1)
    gi = jnp.dot(agg, wih_ref[...], preferred_element_type=jnp.float32) + bih_ref[...]
    gh = jnp.dot(x_ref[...], whh_ref[...], preferred_element_type=jnp.float32) + bhh_ref[...]
    i_r, i_z, i_n = gi[:, :H], gi[:, H:2 * H], gi[:, 2 * H:]
    h_r, h_z, h_n = gh[:, :H], gh[:, H:2 * H], gh[:, 2 * H:]
    r = jax.nn.sigmoid(i_r + h_r)
    z = jax.nn.sigmoid(i_z + h_z)
    n = jnp.tanh(i_n + r * h_n)
    o_ref[...] = (1.0 - z) * n + z * x_ref[...]


def _gru(agg_chunks, x, wih_t, whh_t, bih, bhh):
    return pl.pallas_call(
        _gru_kernel,
        grid=(GRID_N,),
        in_specs=[
            pl.BlockSpec((NB, CW), lambda i: (i, 0)),
            pl.BlockSpec((NB, CW), lambda i: (i, 0)),
            pl.BlockSpec((NB, CW), lambda i: (i, 0)),
            pl.BlockSpec((NB, CW), lambda i: (i, 0)),
            pl.BlockSpec((NB, H), lambda i: (i, 0)),
            pl.BlockSpec((H, 3 * H), lambda i: (0, 0)),
            pl.BlockSpec((H, 3 * H), lambda i: (0, 0)),
            pl.BlockSpec((1, 3 * H), lambda i: (0, 0)),
            pl.BlockSpec((1, 3 * H), lambda i: (0, 0)),
        ],
        out_specs=pl.BlockSpec((NB, H), lambda i: (i, 0)),
        out_shape=jax.ShapeDtypeStruct((N, H), jnp.float32),
    )(*agg_chunks, x, wih_t, whh_t, bih.reshape(1, 3 * H), bhh.reshape(1, 3 * H))


def _attn_w_kernel(qe_ref, k_ref, v_ref, m_ref, we_ref):
    s = jnp.dot(qe_ref[...] * k_ref[...], m_ref[...],
                preferred_element_type=jnp.float32) * (1.0 / math.sqrt(32.0))
    e = jnp.exp(s)
    we_ref[...] = jnp.concatenate([e * v_ref[...], e], axis=1)


def _attn_weights(qexp, k, v, headmask):
    return pl.pallas_call(
        _attn_w_kernel,
        grid=(GRID_N,),
        in_specs=[
            pl.BlockSpec((NB, H), lambda i: (i, 0)),
            pl.BlockSpec((NB, H), lambda i: (i, 0)),
            pl.BlockSpec((NB, H), lambda i: (i, 0)),
            pl.BlockSpec((H, H), lambda i: (0, 0)),
        ],
        out_specs=pl.BlockSpec((NB, 2 * H), lambda i: (i, 0)),
        out_shape=jax.ShapeDtypeStruct((NP, 2 * H), jnp.float32),
    )(qexp, k, v, headmask)


NP = 65536
WNODES = NP // 32
SB = 128
NBAT = WNODES // SB
BROWS = NP // SB
BACC = 512


def _sc_attnsum_body(we_hbm, bat_hbm, zb_hbm, out0, out1,
                     bat_blk, ubuf, acc, sem0, sem1, *ssems):
    c = lax.axis_index("c")
    s = lax.axis_index("s")
    w = c * 16 + s
    zr = BACC // 16
    pltpu.sync_copy(zb_hbm.at[pl.ds(s * zr, zr)], acc.at[pl.ds(s * zr, zr)])
    pltpu.sync_copy(bat_hbm.at[pl.ds(w * NBAT, NBAT)], bat_blk)
    plsc.subcore_barrier()
    usems = (sem0, sem1)

    def _load(j):
        pltpu.async_copy(
            we_hbm.at[pl.ds(w * WNODES + j * SB, SB)], ubuf.at[j % 2],
            usems[j % 2])

    _load(0)
    for j in range(NBAT):
        b = j % 2
        if j + 1 < NBAT:
            if j >= 1:
                pltpu.make_async_copy(
                    ubuf.at[1 - b], acc.at[bat_blk.at[j]], ssems[1 - b]).wait()
            _load(j + 1)
        pltpu.make_async_copy(
            we_hbm.at[pl.ds(w * WNODES + j * SB, SB)], ubuf.at[b],
            usems[b]).wait()
        pltpu.async_copy(ubuf.at[b], acc.at[bat_blk.at[j]], ssems[b],
                         add=True)
    for b in range(2):
        pltpu.make_async_copy(ubuf.at[b], acc.at[bat_blk.at[b]], ssems[b]).wait()
    plsc.subcore_barrier()

    @pl.when(c == 0)
    def _w0():
        pltpu.sync_copy(acc.at[pl.ds(s * zr, zr)], out0.at[pl.ds(s * zr, zr)])

    @pl.when(c == 1)
    def _w1():
        pltpu.sync_copy(acc.at[pl.ds(s * zr, zr)], out1.at[pl.ds(s * zr, zr)])


GROWS = NP // 128


def _sc_gather_body(tab_hbm, idx_hbm, out_hbm, idx_blk, gbuf, g0, g1, s0, s1):
    c = lax.axis_index("c")
    s = lax.axis_index("s")
    w = c * 16 + s
    gsems = (g0, g1)
    osems = (s0, s1)
    nb = GROWS // 32
    pltpu.sync_copy(idx_hbm.at[pl.ds(w * nb, nb)], idx_blk)

    def _g(j):
        pltpu.async_copy(tab_hbm.at[idx_blk.at[j]], gbuf.at[j % 2], gsems[j % 2])

    def _dst(j):
        return out_hbm.at[pl.ds((w * nb + j) * 128, 128)]

    _g(0)
    for j in range(nb):
        b = j % 2
        if j + 1 < nb:
            if j >= 1:
                pltpu.make_async_copy(gbuf.at[1 - b], _dst(j), osems[1 - b]).wait()
            _g(j + 1)
        pltpu.make_async_copy(tab_hbm.at[idx_blk.at[j]], gbuf.at[b],
                              gsems[b]).wait()
        pltpu.async_copy(gbuf.at[b], _dst(j), osems[b])
    for b in range(2):
        pltpu.make_async_copy(gbuf.at[b], _dst(b), osems[b]).wait()


def _sc_gather(table, idx2d):
    mesh = plsc.VectorSubcoreMesh(core_axis_name="c", subcore_axis_name="s")
    out_type = jax.ShapeDtypeStruct((NP, H), jnp.float32)
    scratch = [
        pltpu.VMEM((GROWS // 32, 128), jnp.int32),
        pltpu.VMEM((2, 128, H), jnp.float32),
        pltpu.SemaphoreType.DMA,
        pltpu.SemaphoreType.DMA,
        pltpu.SemaphoreType.DMA,
        pltpu.SemaphoreType.DMA,
    ]
    f = pl.kernel(_sc_gather_body, out_type=out_type, mesh=mesh,
                  scratch_types=scratch,
                  compiler_params=pltpu.CompilerParams(use_tc_tiling_on_sc=True))
    return f(table, idx2d)


def _sc_attnsum(we, bat2d, zb):
    mesh = plsc.VectorSubcoreMesh(core_axis_name="c", subcore_axis_name="s")
    out_type = [jax.ShapeDtypeStruct((BACC, 2 * H), jnp.float32) for _ in range(2)]
    scratch = [
        pltpu.VMEM((NBAT, SB), jnp.int32),
        pltpu.VMEM((2, SB, 2 * H), jnp.float32),
        pltpu.VMEM_SHARED((BACC, 2 * H), jnp.float32),
        pltpu.SemaphoreType.DMA,
        pltpu.SemaphoreType.DMA,
        pltpu.SemaphoreType.DMA,
        pltpu.SemaphoreType.DMA,
    ]
    f = pl.kernel(_sc_attnsum_body, out_type=out_type, mesh=mesh,
                  scratch_types=scratch,
                  compiler_params=pltpu.CompilerParams(use_tc_tiling_on_sc=False))
    return f(we, bat2d, zb)


def _head_kernel(a0_ref, a1_ref, wo_ref, bo_ref, o_ref):
    t = a0_ref[...] + a1_ref[...]
    es = t[:B, H:]
    ctx = t[:B, :H] / jnp.where(es > 0.0, es, 1.0)
    o_ref[...] = jnp.dot(ctx, wo_ref[...], preferred_element_type=jnp.float32) + bo_ref[...]


def _head(acc0, acc1, wo_t, bo):
    return pl.pallas_call(
        _head_kernel,
        in_specs=[
            pl.BlockSpec((BACC, 2 * H), lambda: (0, 0)),
            pl.BlockSpec((BACC, 2 * H), lambda: (0, 0)),
            pl.BlockSpec((H, H), lambda: (0, 0)),
            pl.BlockSpec((1, H), lambda: (0, 0)),
        ],
        out_specs=pl.BlockSpec((B, H), lambda: (0, 0)),
        out_shape=jax.ShapeDtypeStruct((B, H), jnp.float32),
    )(acc0, acc1, wo_t, bo.reshape(1, H))


def _logits_kernel(f_ref, w_ref, b_ref, o_ref):
    o_ref[...] = jnp.dot(f_ref[...], w_ref[...], preferred_element_type=jnp.float32) + b_ref[...]


def _logits(final, lin_w_t, lin_b):
    V = lin_w_t.shape[1]
    CB = 1280
    grid = (V + CB - 1) // CB
    return pl.pallas_call(
        _logits_kernel,
        grid=(grid,),
        in_specs=[
            pl.BlockSpec((B, H), lambda j: (0, 0)),
            pl.BlockSpec((H, CB), lambda j: (0, j)),
            pl.BlockSpec((1, CB), lambda j: (0, j)),
        ],
        out_specs=pl.BlockSpec((B, CB), lambda j: (0, j)),
        out_shape=jax.ShapeDtypeStruct((B, V), jnp.float32),
    )(final, lin_w_t, lin_b.reshape(1, V))


def kernel(node_idx, time_feat, dwell_feat, edge_index, batch, embed, W_in, b_in,
           ggnn_w, gru_w_ih, gru_w_hh, gru_b_ih, gru_b_hh,
           attn_in_w, attn_in_b, attn_out_w, attn_out_b, lin_w, lin_b):
    x_emb = jnp.take(embed, node_idx, axis=0)
    wtd = W_in.T[126:128, :]
    x = _inproj(x_emb, time_feat, dwell_feat, W_in.T[:126, :], wtd, b_in)

    npad = E_PAD - E
    src2d = jnp.concatenate(
        [edge_index[0], jnp.zeros((npad,), edge_index.dtype)]).reshape(EROWS, EB)
    dst2d = jnp.concatenate(
        [edge_index[1], jnp.full((npad,), N, edge_index.dtype)]).reshape(EROWS, EB)
    zeros = jnp.zeros((NACC, CW), jnp.float32)
    for i in range(2):
        m_chunks = _mm4(x, ggnn_w[i])
        agg_chunks = _sc_segsum(src2d, dst2d, zeros, m_chunks)
        x = _gru(agg_chunks, x, gru_w_ih.T, gru_w_hh.T, gru_b_ih, gru_b_hh)

    qkv = _mm_bias(x, attn_in_w.T, attn_in_b)
    q = qkv[:, :H]
    k = qkv[:, H:2 * H]
    v = qkv[:, 2 * H:]
    lengths = jnp.bincount(batch, length=B)
    last_flat = jnp.cumsum(lengths) - 1
    qexp = jnp.take(jnp.take(q, last_flat, axis=0), batch, axis=0)
    headmask = jnp.repeat(jnp.eye(4, dtype=jnp.float32), 32, axis=0)
    headmask = jnp.repeat(headmask, 32, axis=1)
    we = _attn_weights(qexp, k, v, headmask)
    bat2d = jnp.concatenate(
        [batch.astype(jnp.int32), jnp.full((NP - N,), B, jnp.int32)]).reshape(BROWS, SB)
    zb = jnp.zeros((BACC, 2 * H), jnp.float32)
    acc0, acc1 = _sc_attnsum(we, bat2d, zb)
    final = _head(acc0, acc1, attn_out_w.T, attn_out_b)
    return _logits(final, lin_w.T, lin_b)


# speedup vs baseline: 1.3392x; 1.0308x over previous
import functools
import math

import jax
import jax.numpy as jnp
from jax import lax
from jax.experimental import pallas as pl
from jax.experimental.pallas import tpu as pltpu
from jax.experimental.pallas import tpu_sc as plsc

N = 50000
B = 500
H = 128
E = 800000
NB = 1000
GRID_N = N // NB

CW = 32
EB = 128
E_PAD = 802816
EROWS = E_PAD // EB
TROWS = EROWS // 16
GB = 56
NBLK = TROWS // GB
NBUF = 4
NQ = GB // NBUF
NACC = 50048
ZR = NACC // 16


def _sc_segsum_body(src_hbm, dst_hbm, zeros_hbm, m0, m1, m2, m3,
                    a0, a1, a2, a3,
                    src_blk, dst_blk, rows, acc, *sems):
    gsem = sems[:NBUF]
    ssem = sems[NBUF:]
    c = lax.axis_index("c")
    s = lax.axis_index("s")

    def process(m_ref, a_ref):
        pltpu.sync_copy(zeros_hbm.at[pl.ds(s * ZR, ZR)], acc.at[pl.ds(s * ZR, ZR)])
        plsc.subcore_barrier()

        def blk_body(blk, carry):
            @pl.when(blk > 0)
            def _drain():
                for b in range(NBUF):
                    pltpu.make_async_copy(
                        rows.at[b], acc.at[dst_blk.at[b]], ssem[b]).wait()

            base = s * TROWS + blk * GB
            pltpu.sync_copy(src_hbm.at[pl.ds(base, GB)], src_blk)
            pltpu.sync_copy(dst_hbm.at[pl.ds(base, GB)], dst_blk)
            for gg in range(NQ):
                for b in range(NBUF):
                    r = gg * NBUF + b
                    if gg > 0:
                        pltpu.make_async_copy(
                            rows.at[b], acc.at[dst_blk.at[r]], ssem[b]).wait()
                    pltpu.async_copy(m_ref.at[src_blk.at[r]], rows.at[b], gsem[b])
                for b in range(NBUF):
                    r = gg * NBUF + b
                    pltpu.make_async_copy(
                        m_ref.at[src_blk.at[r]], rows.at[b], gsem[b]).wait()
                    pltpu.async_copy(rows.at[b], acc.at[dst_blk.at[r]], ssem[b],
                                     add=True)
            return carry

        lax.fori_loop(0, NBLK, blk_body, 0)
        for b in range(NBUF):
            pltpu.make_async_copy(
                rows.at[b], acc.at[dst_blk.at[b]], ssem[b]).wait()
        plsc.subcore_barrier()
        pltpu.sync_copy(acc.at[pl.ds(s * ZR, ZR)], a_ref.at[pl.ds(s * ZR, ZR)])
        plsc.subcore_barrier()

    for cc in range(2):
        @pl.when(c == 0)
        def _c0():
            process((m0, m1)[cc], (a0, a1)[cc])

        @pl.when(c == 1)
        def _c1():
            process((m2, m3)[cc], (a2, a3)[cc])


def _sc_segsum(src2d, dst2d, zeros, m_chunks):
    mesh = plsc.VectorSubcoreMesh(core_axis_name="c", subcore_axis_name="s")
    out_type = [jax.ShapeDtypeStruct((NACC, CW), jnp.float32) for _ in range(4)]
    scratch = [
        pltpu.VMEM((GB, EB), jnp.int32),
        pltpu.VMEM((GB, EB), jnp.int32),
        pltpu.VMEM((NBUF, EB, CW), jnp.float32),
        pltpu.VMEM_SHARED((NACC, CW), jnp.float32),
    ] + [pltpu.SemaphoreType.DMA] * (2 * NBUF)
    f = pl.kernel(_sc_segsum_body, out_type=out_type, mesh=mesh,
                  scratch_types=scratch,
                  compiler_params=pltpu.CompilerParams(use_tc_tiling_on_sc=False))
    return f(src2d, dst2d, zeros, *m_chunks)


def _mm_bias_kernel(x_ref, w_ref, b_ref, o_ref, *, act):
    y = jnp.dot(x_ref[...], w_ref[...], preferred_element_type=jnp.float32) + b_ref[...]
    if act == "relu":
        y = jnp.maximum(y, 0.0)
    o_ref[...] = y


def _inproj_kernel(x_ref, tf_ref, df_ref, w_ref, wtd_ref, b_ref, o_ref):
    y = jnp.dot(x_ref[...], w_ref[...], preferred_element_type=jnp.float32)
    y = y + tf_ref[...] * wtd_ref[0:1, :] + df_ref[...] * wtd_ref[1:2, :]
    o_ref[...] = jnp.maximum(y + b_ref[...], 0.0)


def _inproj(x_emb, tf, df, w_t, wtd, b):
    K = x_emb.shape[1]
    return pl.pallas_call(
        _inproj_kernel,
        grid=(GRID_N,),
        in_specs=[
            pl.BlockSpec((NB, K), lambda i: (i, 0)),
            pl.BlockSpec((NB, 1), lambda i: (i, 0)),
            pl.BlockSpec((NB, 1), lambda i: (i, 0)),
            pl.BlockSpec((K, H), lambda i: (0, 0)),
            pl.BlockSpec((2, H), lambda i: (0, 0)),
            pl.BlockSpec((1, H), lambda i: (0, 0)),
        ],
        out_specs=pl.BlockSpec((NB, H), lambda i: (i, 0)),
        out_shape=jax.ShapeDtypeStruct((N, H), jnp.float32),
    )(x_emb, tf, df, w_t, wtd, b.reshape(1, H))


def _mm_bias(x, w_t, b, act="none"):
    K = x.shape[1]
    M = w_t.shape[1]
    return pl.pallas_call(
        functools.partial(_mm_bias_kernel, act=act),
        grid=(GRID_N,),
        in_specs=[
            pl.BlockSpec((NB, K), lambda i: (i, 0)),
            pl.BlockSpec((K, M), lambda i: (0, 0)),
            pl.BlockSpec((1, M), lambda i: (0, 0)),
        ],
        out_specs=pl.BlockSpec((NB, M), lambda i: (i, 0)),
        out_shape=jax.ShapeDtypeStruct((N, M), jnp.float32),
    )(x, w_t, b.reshape(1, M))


def _mm4_kernel(x_ref, w_ref, o0, o1, o2, o3):
    y = jnp.dot(x_ref[...], w_ref[...], preferred_element_type=jnp.float32)
    for i, o in enumerate((o0, o1, o2, o3)):
        o[...] = y[:, i * CW:(i + 1) * CW]


def _mm4(x, w):
    return pl.pallas_call(
        _mm4_kernel,
        grid=(GRID_N,),
        in_specs=[
            pl.BlockSpec((NB, H), lambda i: (i, 0)),
            pl.BlockSpec((H, H), lambda i: (0, 0)),
        ],
        out_specs=[pl.BlockSpec((NB, CW), lambda i: (i, 0)) for _ in range(4)],
        out_shape=[jax.ShapeDtypeStruct((N, CW), jnp.float32) for _ in range(4)],
    )(x, w)


def _gru_kernel(a0, a1, a2, a3, x_ref, wih_ref, whh_ref, bih_ref, bhh_ref, o_ref):
    agg = jnp.concatenate([a0[...], a1[...], a2[...], a3[...]], axis=1)
    gi = jnp.dot(agg, wih_ref[...], preferred_element_type=jnp.float32) + bih_ref[...]
    gh = jnp.dot(x_ref[...], whh_ref[...], preferred_element_type=jnp.float32) + bhh_ref[...]
    i_r, i_z, i_n = gi[:, :H], gi[:, H:2 * H], gi[:, 2 * H:]
    h_r, h_z, h_n = gh[:, :H], gh[:, H:2 * H], gh[:, 2 * H:]
    r = jax.nn.sigmoid(i_r + h_r)
    z = jax.nn.sigmoid(i_z + h_z)
    n = jnp.tanh(i_n + r * h_n)
    o_ref[...] = (1.0 - z) * n + z * x_ref[...]


def _gru(agg_chunks, x, wih_t, whh_t, bih, bhh):
    return pl.pallas_call(
        _gru_kernel,
        grid=(GRID_N,),
        in_specs=[
            pl.BlockSpec((NB, CW), lambda i: (i, 0)),
            pl.BlockSpec((NB, CW), lambda i: (i, 0)),
            pl.BlockSpec((NB, CW), lambda i: (i, 0)),
            pl.BlockSpec((NB, CW), lambda i: (i, 0)),
            pl.BlockSpec((NB, H), lambda i: (i, 0)),
            pl.BlockSpec((H, 3 * H), lambda i: (0, 0)),
            pl.BlockSpec((H, 3 * H), lambda i: (0, 0)),
            pl.BlockSpec((1, 3 * H), lambda i: (0, 0)),
            pl.BlockSpec((1, 3 * H), lambda i: (0, 0)),
        ],
        out_specs=pl.BlockSpec((NB, H), lambda i: (i, 0)),
        out_shape=jax.ShapeDtypeStruct((N, H), jnp.float32),
    )(*agg_chunks, x, wih_t, whh_t, bih.reshape(1, 3 * H), bhh.reshape(1, 3 * H))


def _attn_w_kernel(qe_ref, k_ref, v_ref, m_ref, wv_ref, e_ref):
    s = jnp.dot(qe_ref[...] * k_ref[...], m_ref[...],
                preferred_element_type=jnp.float32) * (1.0 / math.sqrt(32.0))
    e = jnp.exp(s)
    wv_ref[...] = e * v_ref[...]
    e_ref[...] = e


def _attn_weights(qexp, k, v, headmask):
    return pl.pallas_call(
        _attn_w_kernel,
        grid=(GRID_N,),
        in_specs=[
            pl.BlockSpec((NB, H), lambda i: (i, 0)),
            pl.BlockSpec((NB, H), lambda i: (i, 0)),
            pl.BlockSpec((NB, H), lambda i: (i, 0)),
            pl.BlockSpec((H, H), lambda i: (0, 0)),
        ],
        out_specs=[
            pl.BlockSpec((NB, H), lambda i: (i, 0)),
            pl.BlockSpec((NB, H), lambda i: (i, 0)),
        ],
        out_shape=[
            jax.ShapeDtypeStruct((NP, H), jnp.float32),
            jax.ShapeDtypeStruct((NP, H), jnp.float32),
        ],
    )(qexp, k, v, headmask)


NP = 50176
WNODES = NP // 32
SB = 112
NBAT = WNODES // SB
BROWS = NP // SB
BACC = 512


def _sc_attnsum_body(wv_hbm, e_hbm, bat_hbm, zb_hbm, ow0, oe0, ow1, oe1,
                     bat_blk, ubw, ube, accw, acce, *sems):
    c = lax.axis_index("c")
    s = lax.axis_index("s")
    w = c * 16 + s
    zr = BACC // 16
    pltpu.sync_copy(zb_hbm.at[pl.ds(s * zr, zr)], accw.at[pl.ds(s * zr, zr)])
    pltpu.sync_copy(zb_hbm.at[pl.ds(s * zr, zr)], acce.at[pl.ds(s * zr, zr)])
    pltpu.sync_copy(bat_hbm.at[pl.ds(w * NBAT, NBAT)], bat_blk)
    plsc.subcore_barrier()
    uw = sems[0:2]
    ue = sems[2:4]
    sw = sems[4:6]
    se = sems[6:8]

    def _load(j):
        b = j % 2
        sl = pl.ds(w * WNODES + j * SB, SB)
        pltpu.async_copy(wv_hbm.at[sl], ubw.at[b], uw[b])
        pltpu.async_copy(e_hbm.at[sl], ube.at[b], ue[b])

    _load(0)
    for j in range(NBAT):
        b = j % 2
        sl = pl.ds(w * WNODES + j * SB, SB)
        if j + 1 < NBAT:
            if j >= 1:
                pltpu.make_async_copy(
                    ubw.at[1 - b], accw.at[bat_blk.at[j]], sw[1 - b]).wait()
                pltpu.make_async_copy(
                    ube.at[1 - b], acce.at[bat_blk.at[j]], se[1 - b]).wait()
            _load(j + 1)
        pltpu.make_async_copy(wv_hbm.at[sl], ubw.at[b], uw[b]).wait()
        pltpu.make_async_copy(e_hbm.at[sl], ube.at[b], ue[b]).wait()
        pltpu.async_copy(ubw.at[b], accw.at[bat_blk.at[j]], sw[b], add=True)
        pltpu.async_copy(ube.at[b], acce.at[bat_blk.at[j]], se[b], add=True)
    for b in range(2):
        pltpu.make_async_copy(ubw.at[b], accw.at[bat_blk.at[b]], sw[b]).wait()
        pltpu.make_async_copy(ube.at[b], acce.at[bat_blk.at[b]], se[b]).wait()
    plsc.subcore_barrier()

    @pl.when(c == 0)
    def _w0():
        pltpu.sync_copy(accw.at[pl.ds(s * zr, zr)], ow0.at[pl.ds(s * zr, zr)])
        pltpu.sync_copy(acce.at[pl.ds(s * zr, zr)], oe0.at[pl.ds(s * zr, zr)])

    @pl.when(c == 1)
    def _w1():
        pltpu.sync_copy(accw.at[pl.ds(s * zr, zr)], ow1.at[pl.ds(s * zr, zr)])
        pltpu.sync_copy(acce.at[pl.ds(s * zr, zr)], oe1.at[pl.ds(s * zr, zr)])


GROWS = NP // 128


def _sc_gather_body(tab_hbm, idx_hbm, out_hbm, idx_blk, gbuf, g0, g1, s0, s1):
    c = lax.axis_index("c")
    s = lax.axis_index("s")
    w = c * 16 + s
    gsems = (g0, g1)
    osems = (s0, s1)
    nb = GROWS // 32
    pltpu.sync_copy(idx_hbm.at[pl.ds(w * nb, nb)], idx_blk)

    def _g(j):
        pltpu.async_copy(tab_hbm.at[idx_blk.at[j]], gbuf.at[j % 2], gsems[j % 2])

    def _dst(j):
        return out_hbm.at[pl.ds((w * nb + j) * 128, 128)]

    _g(0)
    for j in range(nb):
        b = j % 2
        if j + 1 < nb:
            if j >= 1:
                pltpu.make_async_copy(gbuf.at[1 - b], _dst(j), osems[1 - b]).wait()
            _g(j + 1)
        pltpu.make_async_copy(tab_hbm.at[idx_blk.at[j]], gbuf.at[b],
                              gsems[b]).wait()
        pltpu.async_copy(gbuf.at[b], _dst(j), osems[b])
    for b in range(2):
        pltpu.make_async_copy(gbuf.at[b], _dst(b), osems[b]).wait()


def _sc_gather(table, idx2d):
    mesh = plsc.VectorSubcoreMesh(core_axis_name="c", subcore_axis_name="s")
    out_type = jax.ShapeDtypeStruct((NP, H), jnp.float32)
    scratch = [
        pltpu.VMEM((GROWS // 32, 128), jnp.int32),
        pltpu.VMEM((2, 128, H), jnp.float32),
        pltpu.SemaphoreType.DMA,
        pltpu.SemaphoreType.DMA,
        pltpu.SemaphoreType.DMA,
        pltpu.SemaphoreType.DMA,
    ]
    f = pl.kernel(_sc_gather_body, out_type=out_type, mesh=mesh,
                  scratch_types=scratch,
                  compiler_params=pltpu.CompilerParams(use_tc_tiling_on_sc=True))
    return f(table, idx2d)


def _sc_attnsum(wv, e128, bat2d, zb):
    mesh = plsc.VectorSubcoreMesh(core_axis_name="c", subcore_axis_name="s")
    out_type = [jax.ShapeDtypeStruct((BACC, H), jnp.float32) for _ in range(4)]
    scratch = [
        pltpu.VMEM((NBAT, SB), jnp.int32),
        pltpu.VMEM((2, SB, H), jnp.float32),
        pltpu.VMEM((2, SB, H), jnp.float32),
        pltpu.VMEM_SHARED((BACC, H), jnp.float32),
        pltpu.VMEM_SHARED((BACC, H), jnp.float32),
    ] + [pltpu.SemaphoreType.DMA] * 8
    f = pl.kernel(_sc_attnsum_body, out_type=out_type, mesh=mesh,
                  scratch_types=scratch,
                  compiler_params=pltpu.CompilerParams(use_tc_tiling_on_sc=False))
    return f(wv, e128, bat2d, zb)


def _head_kernel(w0_ref, e0_ref, w1_ref, e1_ref, wo_ref, bo_ref, o_ref):
    wv = w0_ref[...] + w1_ref[...]
    es = e0_ref[...] + e1_ref[...]
    ctx = wv[:B, :] / jnp.where(es[:B, :] > 0.0, es[:B, :], 1.0)
    o_ref[...] = jnp.dot(ctx, wo_ref[...], preferred_element_type=jnp.float32) + bo_ref[...]


def _head(ow0, oe0, ow1, oe1, wo_t, bo):
    return pl.pallas_call(
        _head_kernel,
        in_specs=[
            pl.BlockSpec((BACC, H), lambda: (0, 0)),
            pl.BlockSpec((BACC, H), lambda: (0, 0)),
            pl.BlockSpec((BACC, H), lambda: (0, 0)),
            pl.BlockSpec((BACC, H), lambda: (0, 0)),
            pl.BlockSpec((H, H), lambda: (0, 0)),
            pl.BlockSpec((1, H), lambda: (0, 0)),
        ],
        out_specs=pl.BlockSpec((B, H), lambda: (0, 0)),
        out_shape=jax.ShapeDtypeStruct((B, H), jnp.float32),
    )(ow0, oe0, ow1, oe1, wo_t, bo.reshape(1, H))


def _logits_kernel(f_ref, w_ref, b_ref, o_ref):
    o_ref[...] = jnp.dot(f_ref[...], w_ref[...], preferred_element_type=jnp.float32) + b_ref[...]


def _logits(final, lin_w_t, lin_b):
    V = lin_w_t.shape[1]
    CB = 1280
    grid = (V + CB - 1) // CB
    return pl.pallas_call(
        _logits_kernel,
        grid=(grid,),
        in_specs=[
            pl.BlockSpec((B, H), lambda j: (0, 0)),
            pl.BlockSpec((H, CB), lambda j: (0, j)),
            pl.BlockSpec((1, CB), lambda j: (0, j)),
        ],
        out_specs=pl.BlockSpec((B, CB), lambda j: (0, j)),
        out_shape=jax.ShapeDtypeStruct((B, V), jnp.float32),
    )(final, lin_w_t, lin_b.reshape(1, V))


def kernel(node_idx, time_feat, dwell_feat, edge_index, batch, embed, W_in, b_in,
           ggnn_w, gru_w_ih, gru_w_hh, gru_b_ih, gru_b_hh,
           attn_in_w, attn_in_b, attn_out_w, attn_out_b, lin_w, lin_b):
    x_emb = jnp.take(embed, node_idx, axis=0)
    wtd = W_in.T[126:128, :]
    x = _inproj(x_emb, time_feat, dwell_feat, W_in.T[:126, :], wtd, b_in)

    npad = E_PAD - E
    src2d = jnp.concatenate(
        [edge_index[0], jnp.zeros((npad,), edge_index.dtype)]).reshape(EROWS, EB)
    dst2d = jnp.concatenate(
        [edge_index[1], jnp.full((npad,), N, edge_index.dtype)]).reshape(EROWS, EB)
    zeros = jnp.zeros((NACC, CW), jnp.float32)
    for i in range(2):
        m_chunks = _mm4(x, ggnn_w[i])
        agg_chunks = _sc_segsum(src2d, dst2d, zeros, m_chunks)
        x = _gru(agg_chunks, x, gru_w_ih.T, gru_w_hh.T, gru_b_ih, gru_b_hh)

    qkv = _mm_bias(x, attn_in_w.T, attn_in_b)
    q = qkv[:, :H]
    k = qkv[:, H:2 * H]
    v = qkv[:, 2 * H:]
    lengths = jnp.bincount(batch, length=B)
    last_flat = jnp.cumsum(lengths) - 1
    qexp = jnp.take(jnp.take(q, last_flat, axis=0), batch, axis=0)
    headmask = jnp.repeat(jnp.eye(4, dtype=jnp.float32), 32, axis=0)
    headmask = jnp.repeat(headmask, 32, axis=1)
    wv, e128 = _attn_weights(qexp, k, v, headmask)
    bat2d = jnp.concatenate(
        [batch.astype(jnp.int32), jnp.full((NP - N,), B, jnp.int32)]).reshape(BROWS, SB)
    zb = jnp.zeros((BACC, H), jnp.float32)
    ow0, oe0, ow1, oe1 = _sc_attnsum(wv, e128, bat2d, zb)
    final = _head(ow0, oe0, ow1, oe1, attn_out_w.T, attn_out_b)
    return _logits(final, lin_w.T, lin_b)
